# baseline (speedup 1.0000x reference)
import numpy as np

import jax
import jax.numpy as jnp
from jax.experimental import pallas as pl
from jax.experimental.pallas import tpu as pltpu

_NUM_TRAIN_TIMESTEPS = 1000
_BETA_START = 0.0001
_BETA_END = 0.02


def _schedule_tables():
    betas = np.linspace(_BETA_START, _BETA_END, _NUM_TRAIN_TIMESTEPS,
                        dtype=np.float32)
    alphas_cumprod = np.cumprod(1.0 - betas, axis=0, dtype=np.float32)
    sqrt_a = np.sqrt(alphas_cumprod).astype(np.float32)
    sqrt_oma = np.sqrt(1.0 - alphas_cumprod).astype(np.float32)
    return sqrt_a, sqrt_oma


_SQRT_A, _SQRT_OMA = _schedule_tables()


_BB = 2


def _fma_body(ts_ref, ta_ref, tb_ref, x_ref, n_ref, o_ref):
    i = pl.program_id(0)
    for k in range(_BB):
        t = ts_ref[_BB * i + k]
        a = ta_ref[t]
        s = tb_ref[t]
        o_ref[k] = x_ref[k] * a + n_ref[k] * s


def kernel(x_0, noise, timesteps):
    B, N, C = x_0.shape
    grid = (B // _BB,)
    spec = pl.BlockSpec((_BB, N, C), lambda i, *_: (i, 0, 0))
    return pl.pallas_call(
        _fma_body,
        grid_spec=pltpu.PrefetchScalarGridSpec(
            num_scalar_prefetch=3,
            grid=grid,
            in_specs=[spec, spec],
            out_specs=spec,
        ),
        out_shape=jax.ShapeDtypeStruct((B, N, C), x_0.dtype),
        compiler_params=pltpu.CompilerParams(
            dimension_semantics=("parallel",),
        ),
    )(timesteps.astype(jnp.int32), jnp.asarray(_SQRT_A), jnp.asarray(_SQRT_OMA),
      x_0, noise)


# speedup vs baseline: 1.0024x; 1.0024x over previous
import numpy as np

import jax
import jax.numpy as jnp
from jax.experimental import pallas as pl
from jax.experimental.pallas import tpu as pltpu

_NUM_TRAIN_TIMESTEPS = 1000
_BETA_START = 0.0001
_BETA_END = 0.02


def _schedule_tables():
    betas = np.linspace(_BETA_START, _BETA_END, _NUM_TRAIN_TIMESTEPS,
                        dtype=np.float32)
    alphas_cumprod = np.cumprod(1.0 - betas, axis=0, dtype=np.float32)
    sqrt_a = np.sqrt(alphas_cumprod).astype(np.float32)
    sqrt_oma = np.sqrt(1.0 - alphas_cumprod).astype(np.float32)
    return sqrt_a, sqrt_oma


_SQRT_A, _SQRT_OMA = _schedule_tables()


_BB = 1


def _fma_body(ts_ref, ta_ref, tb_ref, x_ref, n_ref, o_ref):
    i = pl.program_id(0)
    for k in range(_BB):
        t = ts_ref[_BB * i + k]
        a = ta_ref[t]
        s = tb_ref[t]
        o_ref[k] = x_ref[k] * a + n_ref[k] * s


def kernel(x_0, noise, timesteps):
    B, N, C = x_0.shape
    grid = (B // _BB,)
    spec = pl.BlockSpec((_BB, N, C), lambda i, *_: (i, 0, 0))
    return pl.pallas_call(
        _fma_body,
        grid_spec=pltpu.PrefetchScalarGridSpec(
            num_scalar_prefetch=3,
            grid=grid,
            in_specs=[spec, spec],
            out_specs=spec,
        ),
        out_shape=jax.ShapeDtypeStruct((B, N, C), x_0.dtype),
        compiler_params=pltpu.CompilerParams(
            dimension_semantics=("parallel",),
        ),
    )(timesteps.astype(jnp.int32), jnp.asarray(_SQRT_A), jnp.asarray(_SQRT_OMA),
      x_0, noise)
